# baseline (speedup 1.0000x reference)
import functools

import jax
import jax.numpy as jnp
from jax import lax
from jax.experimental import pallas as pl
from jax.experimental.pallas import tpu as pltpu
from jax.experimental.pallas import tpu_sc as plsc

_N = 10000
_E = 320000
_D = 128
_H = 8
_HD = _D // _H
_SCALE = _HD ** (-0.5)

_NC = 2
_NS = 16
_NW = _NC * _NS
_EPW = _E // _NW
_CHUNK = 40
_NCHUNK = _EPW // _CHUNK
_NP = 10240
_RPT = _NP // _NS
_DR = _NP // 16
_DPT = _DR // _NS

_BR = 400
_GRID = _N // _BR


def _lane_permute(x, idx16):
    dn = lax.GatherDimensionNumbers(offset_dims=(), collapsed_slice_dims=(0,),
                                    start_index_map=(0,))
    return lax.gather(x, idx16[:, None], dn, slice_sizes=(1,),
                      mode=lax.GatherScatterMode.PROMISE_IN_BOUNDS)


def _layernorm(xb):
    mu = jnp.mean(xb, axis=-1, keepdims=True)
    xc = xb - mu
    var = jnp.mean(xc * xc, axis=-1, keepdims=True)
    return xc * lax.rsqrt(var + 1e-6)


def _qkv_body(x_ref, c_ref, wa_ref, ba_ref, wq_ref, bq_ref, wkv_ref, bkv_ref,
              mod_ref, q_ref, kv_ref):
    cb = c_ref[...]
    mod = jax.nn.silu(cb) @ wa_ref[...] + ba_ref[...]
    mod_ref[...] = mod
    ln = _layernorm(x_ref[...])
    h = ln * (1.0 + mod[:, _D:2 * _D]) + mod[:, 0:_D]
    q_ref[...] = (h @ wq_ref[...] + bq_ref[...]) * _SCALE
    kv_ref[...] = h @ wkv_ref[...] + bkv_ref[...]


def _stage_a(x, c, Wa, ba, Wq, bq, Wkv, bkv):
    row = lambda i: (i, 0)
    full = lambda i: (0, 0)
    return pl.pallas_call(
        _qkv_body,
        grid=(_GRID,),
        in_specs=[
            pl.BlockSpec((_BR, _D), row),
            pl.BlockSpec((_BR, _D), row),
            pl.BlockSpec((_D, 6 * _D), full),
            pl.BlockSpec((1, 6 * _D), full),
            pl.BlockSpec((_D, _D), full),
            pl.BlockSpec((1, _D), full),
            pl.BlockSpec((_D, 2 * _D), full),
            pl.BlockSpec((1, 2 * _D), full),
        ],
        out_specs=[
            pl.BlockSpec((_BR, 6 * _D), row),
            pl.BlockSpec((_BR, _D), row),
            pl.BlockSpec((_BR, 2 * _D), row),
        ],
        out_shape=[
            jax.ShapeDtypeStruct((_N, 6 * _D), jnp.float32),
            jax.ShapeDtypeStruct((_N, _D), jnp.float32),
            jax.ShapeDtypeStruct((_N, 2 * _D), jnp.float32),
        ],
    )(x, c, Wa, ba, Wq, bq, Wkv, bkv)


def _edge_body(q_hbm, kv_hbm, sd_hbm, num_hbm, den_hbm,
               sdbuf0, sdbuf1, src0, src1, dst0, dst1, dhi0, dhi1,
               dpad0, dpad1, qrows0, qrows1, kvrows0, kvrows1, msg, dmsg,
               sp_num, sp_den,
               sem_sd0, sem_sd1, sem_q0, sem_q1, sem_kv0, sem_kv1):
    cid = lax.axis_index("c")
    sid = lax.axis_index("s")
    wid = cid * _NS + sid
    lane = lax.iota(jnp.int32, 16)
    zero16 = jnp.zeros((16,), jnp.float32)
    rot8 = jnp.bitwise_and(lane + 8, 15)
    hrots = [jnp.bitwise_or(jnp.bitwise_and(lane, 8),
                            jnp.bitwise_and(lane + s, 7)) for s in (4, 2, 1)]
    idxpair = lax.shift_left(jnp.bitwise_and(lane, 1), 3)
    pairmask = [jnp.maximum(
        1 - jnp.abs(lax.shift_right_logical(lane, 1) - hp), 0
    ).astype(jnp.float32) for hp in range(_H // 2)]
    lomask = jnp.clip(8 - lane, 0, 1).astype(jnp.float32)
    himask = 1.0 - lomask
    headmask = jnp.clip(_H - lane, 0, 1).astype(jnp.float32)
    one_f = jnp.ones((16,), jnp.float32)

    sdbuf = (sdbuf0, sdbuf1)
    src_i = (src0, src1)
    dst_i = (dst0, dst1)
    dsthi = (dhi0, dhi1)
    dpad = (dpad0, dpad1)
    qrows = (qrows0, qrows1)
    kvrows = (kvrows0, kvrows1)
    sem_sd = (sem_sd0, sem_sd1)
    sem_q = (sem_q0, sem_q1)
    sem_kv = (sem_kv0, sem_kv1)

    def _zmsg(r, carry):
        for j in range(_D // 16):
            msg[r, pl.ds(j * 16, 16)] = zero16
        return carry
    lax.fori_loop(0, _CHUNK, _zmsg, 0)
    for r in range(_RPT // _CHUNK):
        pltpu.sync_copy(msg, sp_num.at[pl.ds(sid * _RPT + r * _CHUNK, _CHUNK)])
    pltpu.sync_copy(msg.at[pl.ds(0, _DPT)],
                    sp_den.at[pl.ds(sid * _DPT, _DPT)])
    plsc.subcore_barrier()

    base = wid * _EPW

    def _unpack(b):
        for j in (0, 16, 24):
            sd = sdbuf[b][pl.ds(j, 16)]
            dv = jnp.bitwise_and(sd, 16383)
            dst_i[b][pl.ds(j, 16)] = dv
            dpad[b][pl.ds(j, 16)] = dv
            src_i[b][pl.ds(j, 16)] = lax.shift_right_logical(sd, 14)
            dsthi[b][pl.ds(j, 16)] = lax.shift_right_logical(dv, 4)

    def _issue_sd(j, b):
        return pltpu.async_copy(
            sd_hbm.at[pl.ds(base + j * _CHUNK, _CHUNK)],
            sdbuf[b].at[pl.ds(0, _CHUNK)], sem_sd[b])

    def _issue_rows(b):
        pltpu.async_copy(q_hbm.at[dst_i[b]], qrows[b], sem_q[b])
        pltpu.async_copy(kv_hbm.at[src_i[b]], kvrows[b], sem_kv[b])

    def _wait_sd(j, b):
        pltpu.make_async_copy(
            sd_hbm.at[pl.ds(base + j * _CHUNK, _CHUNK)],
            sdbuf[b].at[pl.ds(0, _CHUNK)], sem_sd[b]).wait()

    def _wait_rows(b):
        pltpu.make_async_copy(q_hbm.at[dst_i[b]], qrows[b], sem_q[b]).wait()
        pltpu.make_async_copy(kv_hbm.at[src_i[b]], kvrows[b],
                              sem_kv[b]).wait()

    def _compute(b):
        @plsc.parallel_loop(0, _CHUNK // 8, unroll=2)
        def _group(g):
            gcarry = None
            dstv = dpad[b][pl.ds(g * 8, 16)]
            for r in range(8):
                e = g * 8 + r
                packed = zero16
                for hp in range(_H // 2):
                    pa = (qrows[b][e, pl.ds(hp * 32, 16)] *
                          kvrows[b][e, pl.ds(hp * 32, 16)])
                    pb = (qrows[b][e, pl.ds(hp * 32 + 16, 16)] *
                          kvrows[b][e, pl.ds(hp * 32 + 16, 16)])
                    va = pa + _lane_permute(pa, rot8)
                    vb = pb + _lane_permute(pb, rot8)
                    m = va * lomask + vb * himask
                    for rv in hrots:
                        m = m + _lane_permute(m, rv)
                    packed = packed + _lane_permute(m, idxpair) * pairmask[hp]
                ex = jnp.exp(packed) * headmask
                for h in range(_H):
                    exb = _lane_permute(ex, jnp.full((16,), h, jnp.int32))
                    msg[e, pl.ds(h * 16, 16)] = (
                        kvrows[b][e, pl.ds(_D + h * 16, 16)] * exb)
                dstb = _lane_permute(dstv, jnp.full((16,), r, jnp.int32))
                jlow = jnp.bitwise_and(dstb, 15)
                off8 = lax.shift_left(jnp.bitwise_and(jlow, 1), 3)
                chunk = lax.shift_right_logical(jlow, 1)
                exsh = _lane_permute(ex, jnp.bitwise_and(lane - off8, 15))
                csel = jnp.maximum(
                    one_f - jnp.abs(chunk - lane).astype(jnp.float32), 0.0)
                for cch in range(_D // 16):
                    eqf = _lane_permute(csel, jnp.full((16,), cch, jnp.int32))
                    dmsg[e, pl.ds(cch * 16, 16)] = exsh * eqf
            return gcarry
        pltpu.sync_copy(msg, sp_num.at[dst_i[b]], add=True)
        pltpu.sync_copy(dmsg, sp_den.at[dsthi[b]], add=True)

    pltpu.sync_copy(sd_hbm.at[pl.ds(base, _CHUNK)],
                    sdbuf[0].at[pl.ds(0, _CHUNK)])
    _unpack(0)
    _issue_rows(0)
    _issue_sd(1, 1)

    def _pair(p, carry):
        _wait_sd(2 * p + 1, 1)
        _unpack(1)
        _issue_rows(1)

        @pl.when(p < _NCHUNK // 2 - 1)
        def _():
            _issue_sd(2 * p + 2, 0)
        _wait_rows(0)
        _compute(0)

        @pl.when(p < _NCHUNK // 2 - 1)
        def _():
            _wait_sd(2 * p + 2, 0)
            _unpack(0)
            _issue_rows(0)
            _issue_sd(2 * p + 3, 1)
        _wait_rows(1)
        _compute(1)
        return carry
    lax.fori_loop(0, _NCHUNK // 2, _pair, 0)

    plsc.subcore_barrier()
    pltpu.sync_copy(sp_num.at[pl.ds(sid * _RPT, _RPT)],
                    num_hbm.at[cid, pl.ds(sid * _RPT, _RPT)])
    pltpu.sync_copy(sp_den.at[pl.ds(sid * _DPT, _DPT)],
                    den_hbm.at[cid, pl.ds(sid * _DPT, _DPT)])


def _stage_b(q, kv, sd):
    mesh = plsc.VectorSubcoreMesh(core_axis_name="c", subcore_axis_name="s",
                                  num_cores=_NC, num_subcores=_NS)
    return pl.kernel(
        _edge_body,
        out_type=[
            jax.ShapeDtypeStruct((_NC, _NP, _D), jnp.float32),
            jax.ShapeDtypeStruct((_NC, _DR, _D), jnp.float32),
        ],
        mesh=mesh,
        scratch_types=(
            [pltpu.VMEM((_CHUNK,), jnp.int32)] * 2 +
            [pltpu.VMEM((_CHUNK,), jnp.int32)] * 6 +
            [pltpu.VMEM((_CHUNK + 16,), jnp.int32)] * 2 +
            [pltpu.VMEM((_CHUNK, _D), jnp.float32)] * 2 +
            [pltpu.VMEM((_CHUNK, 2 * _D), jnp.float32)] * 2 +
            [pltpu.VMEM((_CHUNK, _D), jnp.float32)] * 2 +
            [pltpu.VMEM_SHARED((_NP, _D), jnp.float32),
             pltpu.VMEM_SHARED((_DR, _D), jnp.float32)] +
            [pltpu.SemaphoreType.DMA] * 6
        ),
    )(q, kv, sd)


def _out_body(x_ref, num_ref, den_ref, mod_ref, wo_ref, bo_ref, w1_ref, b1_ref,
              w2_ref, b2_ref, o_ref):
    num = num_ref[0] + num_ref[1]
    den = den_ref[0] + den_ref[1]
    inv = 1.0 / (den + 1e-16)
    hh = lax.broadcasted_iota(jnp.int32, (_H, _D), 0)
    jj = lax.broadcasted_iota(jnp.int32, (_H, _D), 1)
    rep = jnp.where(jj // _HD == hh, 1.0, 0.0)
    agg = num * (inv @ rep)
    attn = agg @ wo_ref[...] + bo_ref[...]
    mod = mod_ref[...]
    gate_msa = mod[:, 2 * _D:3 * _D]
    shift_mlp = mod[:, 3 * _D:4 * _D]
    scale_mlp = mod[:, 4 * _D:5 * _D]
    gate_mlp = mod[:, 5 * _D:6 * _D]
    x1 = x_ref[...] + gate_msa * attn
    h2 = _layernorm(x1) * (1.0 + scale_mlp) + shift_mlp
    g = h2 @ w1_ref[...] + b1_ref[...]
    gg = 0.5 * g * (1.0 + lax.erf(g * (2.0 ** -0.5)))
    mlp = gg @ w2_ref[...] + b2_ref[...]
    o_ref[...] = x1 + gate_mlp * mlp


def _stage_c(x, num, den, mod, Wo, bo, W1, b1, W2, b2):
    row = lambda i: (i, 0)
    full = lambda i: (0, 0)
    return pl.pallas_call(
        _out_body,
        grid=(_GRID,),
        in_specs=[
            pl.BlockSpec((_BR, _D), row),
            pl.BlockSpec((_NC, _BR, _D), lambda i: (0, i, 0)),
            pl.BlockSpec((_NC, _BR, _H), lambda i: (0, i, 0)),
            pl.BlockSpec((_BR, 6 * _D), row),
            pl.BlockSpec((_D, _D), full),
            pl.BlockSpec((1, _D), full),
            pl.BlockSpec((_D, 4 * _D), full),
            pl.BlockSpec((1, 4 * _D), full),
            pl.BlockSpec((4 * _D, _D), full),
            pl.BlockSpec((1, _D), full),
        ],
        out_specs=pl.BlockSpec((_BR, _D), row),
        out_shape=jax.ShapeDtypeStruct((_N, _D), jnp.float32),
    )(x, num, den, mod, Wo, bo, W1, b1, W2, b2)


def kernel(x, c, edge_index, Wq, bq, Wk, bk, Wv, bv, Wo, bo, W1, b1, W2, b2,
           Wa, ba):
    Wkv = jnp.concatenate([Wk, Wv], axis=1)
    bkv = jnp.concatenate([bk, bv]).reshape(1, 2 * _D)
    mod, q, kv = _stage_a(x, c, Wa, ba.reshape(1, 6 * _D),
                          Wq, bq.reshape(1, _D), Wkv, bkv)
    sd = edge_index[0] * 16384 + edge_index[1]
    num, den = _stage_b(q, kv, sd)
    den = den.reshape(_NC, _NP, _H)
    return _stage_c(x, num, den, mod, Wo, bo.reshape(1, _D),
                    W1, b1.reshape(1, 4 * _D), W2, b2.reshape(1, _D))


# speedup vs baseline: 1.9023x; 1.9023x over previous
import functools

import jax
import jax.numpy as jnp
from jax import lax
from jax.experimental import pallas as pl
from jax.experimental.pallas import tpu as pltpu
from jax.experimental.pallas import tpu_sc as plsc

_N = 10000
_E = 320000
_D = 128
_H = 8
_HD = _D // _H
_SCALE = _HD ** (-0.5)

_NC = 2
_NS = 16
_NW = _NC * _NS
_EPW = _E // _NW
_CHUNK = 40
_NCHUNK = _EPW // _CHUNK
_NP = 10240
_RPT = _NP // _NS
_DR = _NP // 16
_DPT = _DR // _NS

_BR = 400
_GRID = _N // _BR


def _lane_permute(x, idx16):
    dn = lax.GatherDimensionNumbers(offset_dims=(), collapsed_slice_dims=(0,),
                                    start_index_map=(0,))
    return lax.gather(x, idx16[:, None], dn, slice_sizes=(1,),
                      mode=lax.GatherScatterMode.PROMISE_IN_BOUNDS)


def _layernorm(xb):
    mu = jnp.mean(xb, axis=-1, keepdims=True)
    xc = xb - mu
    var = jnp.mean(xc * xc, axis=-1, keepdims=True)
    return xc * lax.rsqrt(var + 1e-6)


def _qkv_body(x_ref, c_ref, wa_ref, ba_ref, wq_ref, bq_ref, wkv_ref, bkv_ref,
              mod_ref, q_ref, kv_ref):
    cb = c_ref[...]
    mod = jax.nn.silu(cb) @ wa_ref[...] + ba_ref[...]
    mod_ref[...] = mod
    ln = _layernorm(x_ref[...])
    h = ln * (1.0 + mod[:, _D:2 * _D]) + mod[:, 0:_D]
    q_ref[...] = (h @ wq_ref[...] + bq_ref[...]) * _SCALE
    kv_ref[...] = h @ wkv_ref[...] + bkv_ref[...]


def _stage_a(x, c, Wa, ba, Wq, bq, Wkv, bkv):
    row = lambda i: (i, 0)
    full = lambda i: (0, 0)
    return pl.pallas_call(
        _qkv_body,
        grid=(_GRID,),
        in_specs=[
            pl.BlockSpec((_BR, _D), row),
            pl.BlockSpec((_BR, _D), row),
            pl.BlockSpec((_D, 6 * _D), full),
            pl.BlockSpec((1, 6 * _D), full),
            pl.BlockSpec((_D, _D), full),
            pl.BlockSpec((1, _D), full),
            pl.BlockSpec((_D, 2 * _D), full),
            pl.BlockSpec((1, 2 * _D), full),
        ],
        out_specs=[
            pl.BlockSpec((_BR, 6 * _D), row),
            pl.BlockSpec((_BR, _D), row),
            pl.BlockSpec((_BR, 2 * _D), row),
        ],
        out_shape=[
            jax.ShapeDtypeStruct((_N, 6 * _D), jnp.float32),
            jax.ShapeDtypeStruct((_N, _D), jnp.float32),
            jax.ShapeDtypeStruct((_N, 2 * _D), jnp.float32),
        ],
    )(x, c, Wa, ba, Wq, bq, Wkv, bkv)


def _edge_body(q_hbm, kv_hbm, sd_hbm, num_hbm, den_hbm,
               sdbuf0, sdbuf1, src0, src1, dst0, dst1, dhi0, dhi1,
               dpad0, dpad1, sdst0, sdst1, sdhi0, sdhi1,
               qrows0, qrows1, kvrows0, kvrows1, msg, dmsg,
               sp_num, sp_den,
               sem_sd0, sem_sd1, sem_q0, sem_q1, sem_kv0, sem_kv1,
               sem_n0, sem_n1, sem_d0, sem_d1):
    cid = lax.axis_index("c")
    sid = lax.axis_index("s")
    wid = cid * _NS + sid
    lane = lax.iota(jnp.int32, 16)
    zero16 = jnp.zeros((16,), jnp.float32)
    rot8 = jnp.bitwise_and(lane + 8, 15)
    hrots = [jnp.bitwise_or(jnp.bitwise_and(lane, 8),
                            jnp.bitwise_and(lane + s, 7)) for s in (4, 2, 1)]
    idxpair = lax.shift_left(jnp.bitwise_and(lane, 1), 3)
    pairmask = [jnp.maximum(
        1 - jnp.abs(lax.shift_right_logical(lane, 1) - hp), 0
    ).astype(jnp.float32) for hp in range(_H // 2)]
    lomask = jnp.clip(8 - lane, 0, 1).astype(jnp.float32)
    himask = 1.0 - lomask
    headmask = jnp.clip(_H - lane, 0, 1).astype(jnp.float32)
    one_f = jnp.ones((16,), jnp.float32)

    sdbuf = (sdbuf0, sdbuf1)
    src_i = (src0, src1)
    dst_i = (dst0, dst1)
    dsthi = (dhi0, dhi1)
    dpad = (dpad0, dpad1)
    sdst = (sdst0, sdst1)
    sdhi = (sdhi0, sdhi1)
    qrows = (qrows0, qrows1)
    kvrows = (kvrows0, kvrows1)
    sem_sd = (sem_sd0, sem_sd1)
    sem_q = (sem_q0, sem_q1)
    sem_kv = (sem_kv0, sem_kv1)
    sem_n = (sem_n0, sem_n1)
    sem_d = (sem_d0, sem_d1)

    def _zmsg(r, carry):
        for j in range(_D // 16):
            msg[r, pl.ds(j * 16, 16)] = zero16
        return carry
    lax.fori_loop(0, _CHUNK, _zmsg, 0)
    for r in range(_RPT // _CHUNK):
        pltpu.sync_copy(msg, sp_num.at[pl.ds(sid * _RPT + r * _CHUNK, _CHUNK)])
    pltpu.sync_copy(msg.at[pl.ds(0, _DPT)],
                    sp_den.at[pl.ds(sid * _DPT, _DPT)])
    plsc.subcore_barrier()

    base = wid * _EPW

    def _unpack(b):
        for j in (0, 16, 24):
            sd = sdbuf[b][pl.ds(j, 16)]
            dv = jnp.bitwise_and(sd, 16383)
            dst_i[b][pl.ds(j, 16)] = dv
            dpad[b][pl.ds(j, 16)] = dv
            src_i[b][pl.ds(j, 16)] = lax.shift_right_logical(sd, 14)
            dsthi[b][pl.ds(j, 16)] = lax.shift_right_logical(dv, 4)

    def _issue_sd(j, b):
        return pltpu.async_copy(
            sd_hbm.at[pl.ds(base + j * _CHUNK, _CHUNK)],
            sdbuf[b].at[pl.ds(0, _CHUNK)], sem_sd[b])

    def _issue_rows(b):
        pltpu.async_copy(q_hbm.at[dst_i[b]], qrows[b], sem_q[b])
        pltpu.async_copy(kv_hbm.at[src_i[b]], kvrows[b], sem_kv[b])

    def _wait_sd(j, b):
        pltpu.make_async_copy(
            sd_hbm.at[pl.ds(base + j * _CHUNK, _CHUNK)],
            sdbuf[b].at[pl.ds(0, _CHUNK)], sem_sd[b]).wait()

    def _wait_rows(b):
        pltpu.make_async_copy(q_hbm.at[dst_i[b]], qrows[b], sem_q[b]).wait()
        pltpu.make_async_copy(kv_hbm.at[src_i[b]], kvrows[b],
                              sem_kv[b]).wait()

    def _wait_scatters(b):
        pltpu.make_async_copy(msg, sp_num.at[sdst[b]], sem_n[b]).wait()
        pltpu.make_async_copy(dmsg, sp_den.at[sdhi[b]], sem_d[b]).wait()

    def _compute(b):
        for j in (0, 16, 24):
            sdst[b][pl.ds(j, 16)] = dst_i[b][pl.ds(j, 16)]
            sdhi[b][pl.ds(j, 16)] = dsthi[b][pl.ds(j, 16)]
        def _group(g, gcarry):
            dstv = dpad[b][pl.ds(g * 8, 16)]
            for r in range(8):
                e = g * 8 + r
                packed = zero16
                for hp in range(_H // 2):
                    pa = (qrows[b][e, pl.ds(hp * 32, 16)] *
                          kvrows[b][e, pl.ds(hp * 32, 16)])
                    pb = (qrows[b][e, pl.ds(hp * 32 + 16, 16)] *
                          kvrows[b][e, pl.ds(hp * 32 + 16, 16)])
                    va = pa + _lane_permute(pa, rot8)
                    vb = pb + _lane_permute(pb, rot8)
                    m = va * lomask + vb * himask
                    for rv in hrots:
                        m = m + _lane_permute(m, rv)
                    packed = packed + _lane_permute(m, idxpair) * pairmask[hp]
                ex = jnp.exp(packed) * headmask
                for h in range(_H):
                    exb = _lane_permute(ex, jnp.full((16,), h, jnp.int32))
                    msg[e, pl.ds(h * 16, 16)] = (
                        kvrows[b][e, pl.ds(_D + h * 16, 16)] * exb)
                dstb = _lane_permute(dstv, jnp.full((16,), r, jnp.int32))
                jlow = jnp.bitwise_and(dstb, 15)
                off8 = lax.shift_left(jnp.bitwise_and(jlow, 1), 3)
                chunk = lax.shift_right_logical(jlow, 1)
                exsh = _lane_permute(ex, jnp.bitwise_and(lane - off8, 15))
                csel = jnp.maximum(
                    one_f - jnp.abs(chunk - lane).astype(jnp.float32), 0.0)
                for cch in range(_D // 16):
                    eqf = _lane_permute(csel, jnp.full((16,), cch, jnp.int32))
                    dmsg[e, pl.ds(cch * 16, 16)] = exsh * eqf
            return gcarry
        lax.fori_loop(0, _CHUNK // 8, _group, 0)
        pltpu.async_copy(msg, sp_num.at[sdst[b]], sem_n[b], add=True)
        pltpu.async_copy(dmsg, sp_den.at[sdhi[b]], sem_d[b], add=True)

    pltpu.sync_copy(sd_hbm.at[pl.ds(base, _CHUNK)],
                    sdbuf[0].at[pl.ds(0, _CHUNK)])
    _unpack(0)
    _issue_rows(0)
    _issue_sd(1, 1)

    def _pair(p, carry):
        _wait_sd(2 * p + 1, 1)
        _unpack(1)
        _issue_rows(1)

        @pl.when(p < _NCHUNK // 2 - 1)
        def _():
            _issue_sd(2 * p + 2, 0)
        _wait_rows(0)

        @pl.when(p > 0)
        def _():
            _wait_scatters(1)
        _compute(0)

        @pl.when(p < _NCHUNK // 2 - 1)
        def _():
            _wait_sd(2 * p + 2, 0)
            _unpack(0)
            _issue_rows(0)
            _issue_sd(2 * p + 3, 1)
        _wait_rows(1)
        _wait_scatters(0)
        _compute(1)
        return carry
    lax.fori_loop(0, _NCHUNK // 2, _pair, 0)

    _wait_scatters(1)
    plsc.subcore_barrier()
    pltpu.sync_copy(sp_num.at[pl.ds(sid * _RPT, _RPT)],
                    num_hbm.at[cid, pl.ds(sid * _RPT, _RPT)])
    pltpu.sync_copy(sp_den.at[pl.ds(sid * _DPT, _DPT)],
                    den_hbm.at[cid, pl.ds(sid * _DPT, _DPT)])


def _stage_b(q, kv, sd):
    mesh = plsc.VectorSubcoreMesh(core_axis_name="c", subcore_axis_name="s",
                                  num_cores=_NC, num_subcores=_NS)
    return pl.kernel(
        _edge_body,
        out_type=[
            jax.ShapeDtypeStruct((_NC, _NP, _D), jnp.float32),
            jax.ShapeDtypeStruct((_NC, _DR, _D), jnp.float32),
        ],
        mesh=mesh,
        scratch_types=(
            [pltpu.VMEM((_CHUNK,), jnp.int32)] * 2 +
            [pltpu.VMEM((_CHUNK,), jnp.int32)] * 6 +
            [pltpu.VMEM((_CHUNK + 16,), jnp.int32)] * 2 +
            [pltpu.VMEM((_CHUNK,), jnp.int32)] * 4 +
            [pltpu.VMEM((_CHUNK, _D), jnp.float32)] * 2 +
            [pltpu.VMEM((_CHUNK, 2 * _D), jnp.float32)] * 2 +
            [pltpu.VMEM((_CHUNK, _D), jnp.float32)] * 2 +
            [pltpu.VMEM_SHARED((_NP, _D), jnp.float32),
             pltpu.VMEM_SHARED((_DR, _D), jnp.float32)] +
            [pltpu.SemaphoreType.DMA] * 10
        ),
    )(q, kv, sd)


def _out_body(x_ref, num_ref, den_ref, mod_ref, wo_ref, bo_ref, w1_ref, b1_ref,
              w2_ref, b2_ref, o_ref):
    num = num_ref[0] + num_ref[1]
    den = den_ref[0] + den_ref[1]
    inv = 1.0 / (den + 1e-16)
    hh = lax.broadcasted_iota(jnp.int32, (_H, _D), 0)
    jj = lax.broadcasted_iota(jnp.int32, (_H, _D), 1)
    rep = jnp.where(jj // _HD == hh, 1.0, 0.0)
    agg = num * (inv @ rep)
    attn = agg @ wo_ref[...] + bo_ref[...]
    mod = mod_ref[...]
    gate_msa = mod[:, 2 * _D:3 * _D]
    shift_mlp = mod[:, 3 * _D:4 * _D]
    scale_mlp = mod[:, 4 * _D:5 * _D]
    gate_mlp = mod[:, 5 * _D:6 * _D]
    x1 = x_ref[...] + gate_msa * attn
    h2 = _layernorm(x1) * (1.0 + scale_mlp) + shift_mlp
    g = h2 @ w1_ref[...] + b1_ref[...]
    gg = 0.5 * g * (1.0 + lax.erf(g * (2.0 ** -0.5)))
    mlp = gg @ w2_ref[...] + b2_ref[...]
    o_ref[...] = x1 + gate_mlp * mlp


def _stage_c(x, num, den, mod, Wo, bo, W1, b1, W2, b2):
    row = lambda i: (i, 0)
    full = lambda i: (0, 0)
    return pl.pallas_call(
        _out_body,
        grid=(_GRID,),
        in_specs=[
            pl.BlockSpec((_BR, _D), row),
            pl.BlockSpec((_NC, _BR, _D), lambda i: (0, i, 0)),
            pl.BlockSpec((_NC, _BR, _H), lambda i: (0, i, 0)),
            pl.BlockSpec((_BR, 6 * _D), row),
            pl.BlockSpec((_D, _D), full),
            pl.BlockSpec((1, _D), full),
            pl.BlockSpec((_D, 4 * _D), full),
            pl.BlockSpec((1, 4 * _D), full),
            pl.BlockSpec((4 * _D, _D), full),
            pl.BlockSpec((1, _D), full),
        ],
        out_specs=pl.BlockSpec((_BR, _D), row),
        out_shape=jax.ShapeDtypeStruct((_N, _D), jnp.float32),
    )(x, num, den, mod, Wo, bo, W1, b1, W2, b2)


def kernel(x, c, edge_index, Wq, bq, Wk, bk, Wv, bv, Wo, bo, W1, b1, W2, b2,
           Wa, ba):
    Wkv = jnp.concatenate([Wk, Wv], axis=1)
    bkv = jnp.concatenate([bk, bv]).reshape(1, 2 * _D)
    mod, q, kv = _stage_a(x, c, Wa, ba.reshape(1, 6 * _D),
                          Wq, bq.reshape(1, _D), Wkv, bkv)
    sd = edge_index[0] * 16384 + edge_index[1]
    num, den = _stage_b(q, kv, sd)
    den = den.reshape(_NC, _NP, _H)
    return _stage_c(x, num, den, mod, Wo, bo.reshape(1, _D),
                    W1, b1.reshape(1, 4 * _D), W2, b2.reshape(1, _D))


# speedup vs baseline: 1.9454x; 1.0226x over previous
import functools

import jax
import jax.numpy as jnp
from jax import lax
from jax.experimental import pallas as pl
from jax.experimental.pallas import tpu as pltpu
from jax.experimental.pallas import tpu_sc as plsc

_N = 10000
_E = 320000
_D = 128
_H = 8
_HD = _D // _H
_SCALE = _HD ** (-0.5)

_NC = 2
_NS = 16
_NW = _NC * _NS
_EPW = _E // _NW
_CHUNK = 40
_NCHUNK = _EPW // _CHUNK
_NP = 10240
_RPT = _NP // _NS
_DR = _NP // 16
_DPT = _DR // _NS

_BR = 400
_GRID = _N // _BR


def _lane_permute(x, idx16):
    dn = lax.GatherDimensionNumbers(offset_dims=(), collapsed_slice_dims=(0,),
                                    start_index_map=(0,))
    return lax.gather(x, idx16[:, None], dn, slice_sizes=(1,),
                      mode=lax.GatherScatterMode.PROMISE_IN_BOUNDS)


def _layernorm(xb):
    mu = jnp.mean(xb, axis=-1, keepdims=True)
    xc = xb - mu
    var = jnp.mean(xc * xc, axis=-1, keepdims=True)
    return xc * lax.rsqrt(var + 1e-6)


def _qkv_body(x_ref, c_ref, wa_ref, ba_ref, wq_ref, bq_ref, wkv_ref, bkv_ref,
              mod_ref, q_ref, kv_ref):
    cb = c_ref[...]
    mod = jax.nn.silu(cb) @ wa_ref[...] + ba_ref[...]
    mod_ref[...] = mod
    ln = _layernorm(x_ref[...])
    h = ln * (1.0 + mod[:, _D:2 * _D]) + mod[:, 0:_D]
    q_ref[...] = (h @ wq_ref[...] + bq_ref[...]) * _SCALE
    kv_ref[...] = h @ wkv_ref[...] + bkv_ref[...]


def _stage_a(x, c, Wa, ba, Wq, bq, Wkv, bkv):
    row = lambda i: (i, 0)
    full = lambda i: (0, 0)
    return pl.pallas_call(
        _qkv_body,
        grid=(_GRID,),
        in_specs=[
            pl.BlockSpec((_BR, _D), row),
            pl.BlockSpec((_BR, _D), row),
            pl.BlockSpec((_D, 6 * _D), full),
            pl.BlockSpec((1, 6 * _D), full),
            pl.BlockSpec((_D, _D), full),
            pl.BlockSpec((1, _D), full),
            pl.BlockSpec((_D, 2 * _D), full),
            pl.BlockSpec((1, 2 * _D), full),
        ],
        out_specs=[
            pl.BlockSpec((_BR, 6 * _D), row),
            pl.BlockSpec((_BR, _D), row),
            pl.BlockSpec((_BR, 2 * _D), row),
        ],
        out_shape=[
            jax.ShapeDtypeStruct((_N, 6 * _D), jnp.float32),
            jax.ShapeDtypeStruct((_N, _D), jnp.float32),
            jax.ShapeDtypeStruct((_N, 2 * _D), jnp.float32),
        ],
    )(x, c, Wa, ba, Wq, bq, Wkv, bkv)


def _edge_body(q_hbm, kv_hbm, sd_hbm, num_hbm, den_hbm,
               sdbuf0, sdbuf1, src0, src1, dst0, dst1, dhi0, dhi1,
               dpad0, dpad1, sdst0, sdst1, sdhi0, sdhi1,
               qrows0, qrows1, kvrows0, kvrows1, msg, dmsg,
               sp_num, sp_den,
               sem_sd0, sem_sd1, sem_q0, sem_q1, sem_kv0, sem_kv1,
               sem_n0, sem_n1, sem_d0, sem_d1):
    cid = lax.axis_index("c")
    sid = lax.axis_index("s")
    wid = cid * _NS + sid
    lane = lax.iota(jnp.int32, 16)
    zero16 = jnp.zeros((16,), jnp.float32)
    rot8 = jnp.bitwise_and(lane + 8, 15)
    hrots = [jnp.bitwise_or(jnp.bitwise_and(lane, 8),
                            jnp.bitwise_and(lane + s, 7)) for s in (4, 2, 1)]
    idxpair = lax.shift_left(jnp.bitwise_and(lane, 1), 3)
    pairmask = [jnp.maximum(
        1 - jnp.abs(lax.shift_right_logical(lane, 1) - hp), 0
    ).astype(jnp.float32) for hp in range(_H // 2)]
    lomask = jnp.clip(8 - lane, 0, 1).astype(jnp.float32)
    himask = 1.0 - lomask
    headmask = jnp.clip(_H - lane, 0, 1).astype(jnp.float32)
    one_f = jnp.ones((16,), jnp.float32)

    sdbuf = (sdbuf0, sdbuf1)
    src_i = (src0, src1)
    dst_i = (dst0, dst1)
    dsthi = (dhi0, dhi1)
    dpad = (dpad0, dpad1)
    sdst = (sdst0, sdst1)
    sdhi = (sdhi0, sdhi1)
    qrows = (qrows0, qrows1)
    kvrows = (kvrows0, kvrows1)
    sem_sd = (sem_sd0, sem_sd1)
    sem_q = (sem_q0, sem_q1)
    sem_kv = (sem_kv0, sem_kv1)
    sem_n = (sem_n0, sem_n1)
    sem_d = (sem_d0, sem_d1)

    def _zmsg(r, carry):
        for j in range(_D // 16):
            msg[r, pl.ds(j * 16, 16)] = zero16
        return carry
    lax.fori_loop(0, _CHUNK, _zmsg, 0)
    for r in range(_RPT // _CHUNK):
        pltpu.sync_copy(msg, sp_num.at[pl.ds(sid * _RPT + r * _CHUNK, _CHUNK)])
    pltpu.sync_copy(msg.at[pl.ds(0, _DPT)],
                    sp_den.at[pl.ds(sid * _DPT, _DPT)])
    plsc.subcore_barrier()

    base = wid * _EPW

    def _unpack(b):
        for j in (0, 16, 24):
            sd = sdbuf[b][pl.ds(j, 16)]
            dv = jnp.bitwise_and(sd, 16383)
            dst_i[b][pl.ds(j, 16)] = dv
            dpad[b][pl.ds(j, 16)] = dv
            src_i[b][pl.ds(j, 16)] = lax.shift_right_logical(sd, 14)
            dsthi[b][pl.ds(j, 16)] = lax.shift_right_logical(dv, 4)

    def _issue_sd(j, b):
        return pltpu.async_copy(
            sd_hbm.at[pl.ds(base + j * _CHUNK, _CHUNK)],
            sdbuf[b].at[pl.ds(0, _CHUNK)], sem_sd[b])

    def _issue_rows(b):
        pltpu.async_copy(q_hbm.at[dst_i[b]], qrows[b], sem_q[b])
        pltpu.async_copy(kv_hbm.at[src_i[b]], kvrows[b], sem_kv[b])

    def _wait_sd(j, b):
        pltpu.make_async_copy(
            sd_hbm.at[pl.ds(base + j * _CHUNK, _CHUNK)],
            sdbuf[b].at[pl.ds(0, _CHUNK)], sem_sd[b]).wait()

    def _wait_rows(b):
        pltpu.make_async_copy(q_hbm.at[dst_i[b]], qrows[b], sem_q[b]).wait()
        pltpu.make_async_copy(kv_hbm.at[src_i[b]], kvrows[b],
                              sem_kv[b]).wait()

    def _wait_scatters(b):
        pltpu.make_async_copy(msg, sp_num.at[sdst[b]], sem_n[b]).wait()
        pltpu.make_async_copy(dmsg, sp_den.at[sdhi[b]], sem_d[b]).wait()

    def _compute(b):
        for j in (0, 16, 24):
            sdst[b][pl.ds(j, 16)] = dst_i[b][pl.ds(j, 16)]
            sdhi[b][pl.ds(j, 16)] = dsthi[b][pl.ds(j, 16)]
        def _group(g, gcarry):
            dstv = dpad[b][pl.ds(g * 8, 16)]
            for r in range(8):
                e = g * 8 + r
                packed = zero16
                for hp in range(_H // 2):
                    pa = (qrows[b][e, pl.ds(hp * 32, 16)] *
                          kvrows[b][e, pl.ds(hp * 32, 16)])
                    pb = (qrows[b][e, pl.ds(hp * 32 + 16, 16)] *
                          kvrows[b][e, pl.ds(hp * 32 + 16, 16)])
                    va = pa + _lane_permute(pa, rot8)
                    vb = pb + _lane_permute(pb, rot8)
                    m = va * lomask + vb * himask
                    for rv in hrots:
                        m = m + _lane_permute(m, rv)
                    packed = packed + _lane_permute(m, idxpair) * pairmask[hp]
                ex = jnp.exp(packed) * headmask
                for h in range(_H):
                    exb = _lane_permute(ex, jnp.full((16,), h, jnp.int32))
                    msg[e, pl.ds(h * 16, 16)] = (
                        kvrows[b][e, pl.ds(_D + h * 16, 16)] * exb)
                dstb = _lane_permute(dstv, jnp.full((16,), r, jnp.int32))
                jlow = jnp.bitwise_and(dstb, 15)
                off8 = lax.shift_left(jnp.bitwise_and(jlow, 1), 3)
                chunk = lax.shift_right_logical(jlow, 1)
                exsh = _lane_permute(ex, jnp.bitwise_and(lane - off8, 15))
                for cch in range(_D // 16):
                    eqf = jnp.maximum(
                        one_f - jnp.abs(chunk - cch).astype(jnp.float32), 0.0)
                    dmsg[e, pl.ds(cch * 16, 16)] = exsh * eqf
            return gcarry
        lax.fori_loop(0, _CHUNK // 8, _group, 0)
        pltpu.async_copy(msg, sp_num.at[sdst[b]], sem_n[b], add=True)
        pltpu.async_copy(dmsg, sp_den.at[sdhi[b]], sem_d[b], add=True)

    pltpu.sync_copy(sd_hbm.at[pl.ds(base, _CHUNK)],
                    sdbuf[0].at[pl.ds(0, _CHUNK)])
    _unpack(0)
    _issue_rows(0)
    _issue_sd(1, 1)

    def _pair(p, carry):
        _wait_sd(2 * p + 1, 1)
        _unpack(1)
        _issue_rows(1)

        @pl.when(p < _NCHUNK // 2 - 1)
        def _():
            _issue_sd(2 * p + 2, 0)
        _wait_rows(0)

        @pl.when(p > 0)
        def _():
            _wait_scatters(1)
        _compute(0)

        @pl.when(p < _NCHUNK // 2 - 1)
        def _():
            _wait_sd(2 * p + 2, 0)
            _unpack(0)
            _issue_rows(0)
            _issue_sd(2 * p + 3, 1)
        _wait_rows(1)
        _wait_scatters(0)
        _compute(1)
        return carry
    lax.fori_loop(0, _NCHUNK // 2, _pair, 0)

    _wait_scatters(1)
    plsc.subcore_barrier()
    pltpu.sync_copy(sp_num.at[pl.ds(sid * _RPT, _RPT)],
                    num_hbm.at[cid, pl.ds(sid * _RPT, _RPT)])
    pltpu.sync_copy(sp_den.at[pl.ds(sid * _DPT, _DPT)],
                    den_hbm.at[cid, pl.ds(sid * _DPT, _DPT)])


def _stage_b(q, kv, sd):
    mesh = plsc.VectorSubcoreMesh(core_axis_name="c", subcore_axis_name="s",
                                  num_cores=_NC, num_subcores=_NS)
    return pl.kernel(
        _edge_body,
        out_type=[
            jax.ShapeDtypeStruct((_NC, _NP, _D), jnp.float32),
            jax.ShapeDtypeStruct((_NC, _DR, _D), jnp.float32),
        ],
        mesh=mesh,
        scratch_types=(
            [pltpu.VMEM((_CHUNK,), jnp.int32)] * 2 +
            [pltpu.VMEM((_CHUNK,), jnp.int32)] * 6 +
            [pltpu.VMEM((_CHUNK + 16,), jnp.int32)] * 2 +
            [pltpu.VMEM((_CHUNK,), jnp.int32)] * 4 +
            [pltpu.VMEM((_CHUNK, _D), jnp.float32)] * 2 +
            [pltpu.VMEM((_CHUNK, 2 * _D), jnp.float32)] * 2 +
            [pltpu.VMEM((_CHUNK, _D), jnp.float32)] * 2 +
            [pltpu.VMEM_SHARED((_NP, _D), jnp.float32),
             pltpu.VMEM_SHARED((_DR, _D), jnp.float32)] +
            [pltpu.SemaphoreType.DMA] * 10
        ),
    )(q, kv, sd)


def _out_body(x_ref, num_ref, den_ref, mod_ref, wo_ref, bo_ref, w1_ref, b1_ref,
              w2_ref, b2_ref, o_ref):
    num = num_ref[0] + num_ref[1]
    den = den_ref[0] + den_ref[1]
    inv = 1.0 / (den + 1e-16)
    hh = lax.broadcasted_iota(jnp.int32, (_H, _D), 0)
    jj = lax.broadcasted_iota(jnp.int32, (_H, _D), 1)
    rep = jnp.where(jj // _HD == hh, 1.0, 0.0)
    agg = num * (inv @ rep)
    attn = agg @ wo_ref[...] + bo_ref[...]
    mod = mod_ref[...]
    gate_msa = mod[:, 2 * _D:3 * _D]
    shift_mlp = mod[:, 3 * _D:4 * _D]
    scale_mlp = mod[:, 4 * _D:5 * _D]
    gate_mlp = mod[:, 5 * _D:6 * _D]
    x1 = x_ref[...] + gate_msa * attn
    h2 = _layernorm(x1) * (1.0 + scale_mlp) + shift_mlp
    g = h2 @ w1_ref[...] + b1_ref[...]
    gg = 0.5 * g * (1.0 + lax.erf(g * (2.0 ** -0.5)))
    mlp = gg @ w2_ref[...] + b2_ref[...]
    o_ref[...] = x1 + gate_mlp * mlp


def _stage_c(x, num, den, mod, Wo, bo, W1, b1, W2, b2):
    row = lambda i: (i, 0)
    full = lambda i: (0, 0)
    return pl.pallas_call(
        _out_body,
        grid=(_GRID,),
        in_specs=[
            pl.BlockSpec((_BR, _D), row),
            pl.BlockSpec((_NC, _BR, _D), lambda i: (0, i, 0)),
            pl.BlockSpec((_NC, _BR, _H), lambda i: (0, i, 0)),
            pl.BlockSpec((_BR, 6 * _D), row),
            pl.BlockSpec((_D, _D), full),
            pl.BlockSpec((1, _D), full),
            pl.BlockSpec((_D, 4 * _D), full),
            pl.BlockSpec((1, 4 * _D), full),
            pl.BlockSpec((4 * _D, _D), full),
            pl.BlockSpec((1, _D), full),
        ],
        out_specs=pl.BlockSpec((_BR, _D), row),
        out_shape=jax.ShapeDtypeStruct((_N, _D), jnp.float32),
    )(x, num, den, mod, Wo, bo, W1, b1, W2, b2)


def kernel(x, c, edge_index, Wq, bq, Wk, bk, Wv, bv, Wo, bo, W1, b1, W2, b2,
           Wa, ba):
    Wkv = jnp.concatenate([Wk, Wv], axis=1)
    bkv = jnp.concatenate([bk, bv]).reshape(1, 2 * _D)
    mod, q, kv = _stage_a(x, c, Wa, ba.reshape(1, 6 * _D),
                          Wq, bq.reshape(1, _D), Wkv, bkv)
    sd = edge_index[0] * 16384 + edge_index[1]
    num, den = _stage_b(q, kv, sd)
    den = den.reshape(_NC, _NP, _H)
    return _stage_c(x, num, den, mod, Wo, bo.reshape(1, _D),
                    W1, b1.reshape(1, 4 * _D), W2, b2.reshape(1, _D))
